# Initial kernel scaffold; baseline (speedup 1.0000x reference)
import jax
import jax.numpy as jnp
from jax.experimental import pallas as pl


def kernel(x, edge_index, W1, b1, W2, b2, Wf1, bf1, Wf2, bf2):
    raise NotImplementedError("write your pallas kernel here")


# speedup vs baseline: 8.6294x; 8.6294x over previous
import functools

import jax
import jax.numpy as jnp
from jax import lax
from jax.experimental import pallas as pl
from jax.experimental.pallas import tpu as pltpu
from jax.experimental.pallas import tpu_sc as plsc

N = 10000
E = 320000
D = 128
FFN = 256
BOT = 64

NC = 2
NS = 16
NW = NC * NS
CH = 128
KC = 80
GC = 16
EPW = CH * KC
EPAD = NW * EPW
NACC = 10240
RPS = NACC // NS


def _sc_mesh():
    return plsc.VectorSubcoreMesh(core_axis_name="c", subcore_axis_name="s")


def _deg_kernel(dst_hbm, ones_hbm, zeros_hbm):
    @functools.partial(
        pl.kernel,
        out_type=jax.ShapeDtypeStruct((NC, NACC), jnp.float32),
        mesh=_sc_mesh(),
        scratch_types=[
            pltpu.VMEM_SHARED((NACC,), jnp.float32),
            pltpu.VMEM((KC, CH), jnp.int32),
            pltpu.VMEM((CH,), jnp.float32),
        ],
    )
    def k(dst_ref, ones_ref, zeros_ref, out_ref, acc, idx, ones_v):
        cid = lax.axis_index("c")
        sid = lax.axis_index("s")
        wid = cid * NS + sid
        pltpu.sync_copy(zeros_ref, acc.at[pl.ds(sid * RPS, RPS)])
        pltpu.sync_copy(ones_ref, ones_v)
        pltpu.sync_copy(dst_ref.at[pl.ds(wid * KC, KC)], idx)
        plsc.subcore_barrier()

        def body(j, _):
            pltpu.sync_copy(ones_v, acc.at[idx.at[j]], add=True)
            return 0

        lax.fori_loop(0, KC, body, 0)
        plsc.subcore_barrier()
        pltpu.sync_copy(acc.at[pl.ds(sid * RPS, RPS)],
                        out_ref.at[cid, pl.ds(sid * RPS, RPS)])

    return k(dst_hbm, ones_hbm, zeros_hbm)


def _conv_kernel(g_hbm, src_hbm, dst_hbm, zeros_hbm):
    @functools.partial(
        pl.kernel,
        out_type=jax.ShapeDtypeStruct((NC, NACC, D), jnp.float32),
        mesh=_sc_mesh(),
        scratch_types=[
            pltpu.VMEM_SHARED((NACC, D), jnp.float32),
            pltpu.VMEM((GC, CH), jnp.int32),
            pltpu.VMEM((GC, CH), jnp.int32),
            pltpu.VMEM((CH, D), jnp.float32),
            pltpu.VMEM((CH, D), jnp.float32),
            pltpu.SemaphoreType.DMA,
        ],
    )
    def k(g_ref, src_ref, dst_ref, zeros_ref, out_ref,
          acc, sidx, didx, buf0, buf1, sem):
        cid = lax.axis_index("c")
        sid = lax.axis_index("s")
        wid = cid * NS + sid
        for t in range(10):
            pltpu.sync_copy(zeros_ref.at[pl.ds(t * 64, 64)],
                            acc.at[pl.ds(sid * RPS + t * 64, 64)])
        plsc.subcore_barrier()

        def group(gi, _):
            base = wid * KC + gi * GC
            pltpu.sync_copy(src_ref.at[pl.ds(base, GC)], sidx)
            pltpu.sync_copy(dst_ref.at[pl.ds(base, GC)], didx)

            def body(j, _):
                pltpu.sync_copy(g_ref.at[sidx.at[j]], buf0)
                pltpu.sync_copy(buf0, acc.at[didx.at[j]], add=True)
                return 0

            lax.fori_loop(0, GC, body, 0)
            return 0

        lax.fori_loop(0, KC // GC, group, 0)
        plsc.subcore_barrier()
        pltpu.sync_copy(acc.at[pl.ds(sid * RPS, RPS)],
                        out_ref.at[cid, pl.ds(sid * RPS, RPS)])

    return k(g_hbm, src_hbm, dst_hbm, zeros_hbm)


RB = 1024
GRID = NACC // RB


def _dinv_block(d0, d1):
    return lax.rsqrt(d0 + d1 + 1.0)


def _g1_body(x_ref, d0_ref, d1_ref, w_ref, o_ref):
    dinv = _dinv_block(d0_ref[:], d1_ref[:])
    o_ref[:] = dinv * jnp.dot(x_ref[:], w_ref[:],
                              preferred_element_type=jnp.float32)


def _mid_body(p0_ref, p1_ref, g_ref, d0_ref, d1_ref, b_ref, w_ref, o_ref):
    dinv = _dinv_block(d0_ref[:], d1_ref[:])
    h = jax.nn.relu(dinv * (p0_ref[:] + p1_ref[:] + g_ref[:]) + b_ref[:])
    o_ref[:] = dinv * jnp.dot(h, w_ref[:], preferred_element_type=jnp.float32)


def _head_body(p0_ref, p1_ref, g_ref, d0_ref, d1_ref, b_ref,
               wf1_ref, bf1_ref, wf2_ref, bf2_ref, o_ref):
    dinv = _dinv_block(d0_ref[:], d1_ref[:])
    h = jax.nn.relu(dinv * (p0_ref[:] + p1_ref[:] + g_ref[:]) + b_ref[:])
    f = jax.nn.relu(jnp.dot(h, wf1_ref[:], preferred_element_type=jnp.float32)
                    + bf1_ref[:])
    o_ref[:] = (jnp.dot(f, wf2_ref[:], preferred_element_type=jnp.float32)
                + bf2_ref[:])


def _row_spec(cols):
    return pl.BlockSpec((RB, cols), lambda i: (i, 0))


def _col_spec():
    return pl.BlockSpec((RB, 1), lambda i: (i, 0))


def _full_spec(r, c):
    return pl.BlockSpec((r, c), lambda i: (0, 0))


def _tc_g1(x_pad, d0, d1, W1):
    return pl.pallas_call(
        _g1_body,
        grid=(GRID,),
        in_specs=[_row_spec(D), _col_spec(), _col_spec(), _full_spec(D, D)],
        out_specs=_row_spec(D),
        out_shape=jax.ShapeDtypeStruct((NACC, D), jnp.float32),
    )(x_pad, d0, d1, W1)


def _tc_mid(p0, p1, g, d0, d1, b, W):
    return pl.pallas_call(
        _mid_body,
        grid=(GRID,),
        in_specs=[_row_spec(D), _row_spec(D), _row_spec(D),
                  _col_spec(), _col_spec(),
                  _full_spec(1, D), _full_spec(D, D)],
        out_specs=_row_spec(D),
        out_shape=jax.ShapeDtypeStruct((NACC, D), jnp.float32),
    )(p0, p1, g, d0, d1, b, W)


def _tc_head(p0, p1, g, d0, d1, b, Wf1, bf1, Wf2, bf2):
    return pl.pallas_call(
        _head_body,
        grid=(GRID,),
        in_specs=[_row_spec(D), _row_spec(D), _row_spec(D),
                  _col_spec(), _col_spec(),
                  _full_spec(1, D), _full_spec(D, FFN), _full_spec(1, FFN),
                  _full_spec(FFN, BOT), _full_spec(1, BOT)],
        out_specs=_row_spec(BOT),
        out_shape=jax.ShapeDtypeStruct((NACC, BOT), jnp.float32),
    )(p0, p1, g, d0, d1, b, Wf1, bf1, Wf2, bf2)


def kernel(x, edge_index, W1, b1, W2, b2, Wf1, bf1, Wf2, bf2):
    src = edge_index[0]
    dst = edge_index[1]
    npad = EPAD - E
    src_pad = jnp.concatenate(
        [src, jnp.zeros((npad,), jnp.int32)]).reshape(NW * KC, CH)
    dst_pad = jnp.concatenate(
        [dst, jnp.full((npad,), N, jnp.int32)]).reshape(NW * KC, CH)
    x_pad = jnp.concatenate(
        [x, jnp.zeros((NACC - N, D), jnp.float32)], axis=0)

    ones_ch = jnp.ones((CH,), jnp.float32)
    zeros_deg = jnp.zeros((RPS,), jnp.float32)
    zeros_conv = jnp.zeros((RPS, D), jnp.float32)

    deg_parts = _deg_kernel(dst_pad, ones_ch, zeros_deg)
    d0 = deg_parts[0].reshape(NACC, 1)
    d1 = deg_parts[1].reshape(NACC, 1)

    g1 = _tc_g1(x_pad, d0, d1, W1)
    acc1 = _conv_kernel(g1, src_pad, dst_pad, zeros_conv)
    g2 = _tc_mid(acc1[0], acc1[1], g1, d0, d1, b1.reshape(1, D), W2)
    acc2 = _conv_kernel(g2, src_pad, dst_pad, zeros_conv)
    out = _tc_head(acc2[0], acc2[1], g2, d0, d1, b2.reshape(1, D),
                   Wf1, bf1.reshape(1, FFN), Wf2, bf2.reshape(1, BOT))
    return out[:N]


# speedup vs baseline: 9.9500x; 1.1530x over previous
import functools

import jax
import jax.numpy as jnp
from jax import lax
from jax.experimental import pallas as pl
from jax.experimental.pallas import tpu as pltpu
from jax.experimental.pallas import tpu_sc as plsc

N = 10000
E = 320000
D = 128
FFN = 256
BOT = 64

NC = 2
NS = 16
NW = NC * NS
CH = 128
KC = 80
GC = 16
KCF = 112
KCS = 48
EPW = CH * KC
EPAD = NW * EPW
NACC = 10240
RPS = NACC // NS


def _sc_mesh():
    return plsc.VectorSubcoreMesh(core_axis_name="c", subcore_axis_name="s")


def _deg_kernel(dst_hbm, ones_hbm, zeros_hbm):
    @functools.partial(
        pl.kernel,
        out_type=jax.ShapeDtypeStruct((NC, NACC), jnp.float32),
        mesh=_sc_mesh(),
        scratch_types=[
            pltpu.VMEM_SHARED((NACC,), jnp.float32),
            pltpu.VMEM((KC, CH), jnp.int32),
            pltpu.VMEM((CH,), jnp.float32),
        ],
    )
    def k(dst_ref, ones_ref, zeros_ref, out_ref, acc, idx, ones_v):
        cid = lax.axis_index("c")
        sid = lax.axis_index("s")
        wid = cid * NS + sid
        pltpu.sync_copy(zeros_ref, acc.at[pl.ds(sid * RPS, RPS)])
        pltpu.sync_copy(ones_ref, ones_v)
        pltpu.sync_copy(dst_ref.at[pl.ds(wid * KC, KC)], idx)
        plsc.subcore_barrier()

        def body(j, _):
            pltpu.sync_copy(ones_v, acc.at[idx.at[j]], add=True)
            return 0

        lax.fori_loop(0, KC, body, 0)
        plsc.subcore_barrier()
        pltpu.sync_copy(acc.at[pl.ds(sid * RPS, RPS)],
                        out_ref.at[cid, pl.ds(sid * RPS, RPS)])

    return k(dst_hbm, ones_hbm, zeros_hbm)


def _conv_kernel(g_hbm, src_hbm, dst_hbm, zeros_hbm):
    @functools.partial(
        pl.kernel,
        out_type=jax.ShapeDtypeStruct((NC, NACC, D), jnp.float32),
        mesh=_sc_mesh(),
        scratch_types=[
            pltpu.VMEM_SHARED((NACC, D), jnp.float32),
            pltpu.VMEM((GC, CH), jnp.int32),
            pltpu.VMEM((GC, CH), jnp.int32),
            pltpu.VMEM((CH, D), jnp.float32),
            pltpu.VMEM((CH, D), jnp.float32),
            pltpu.SemaphoreType.DMA,
        ],
    )
    def k(g_ref, src_ref, dst_ref, zeros_ref, out_ref,
          acc, sidx, didx, buf0, buf1, sem):
        cid = lax.axis_index("c")
        sid = lax.axis_index("s")
        wid = cid * NS + sid
        for t in range(10):
            pltpu.sync_copy(zeros_ref.at[pl.ds(t * 64, 64)],
                            acc.at[pl.ds(sid * RPS + t * 64, 64)])
        plsc.subcore_barrier()

        def scat(j, buf):
            pltpu.sync_copy(buf, acc.at[didx.at[j]], add=True)

        row0 = jnp.where(cid == 0, sid * KCF, NS * KCF + sid * KCS)
        ngrp = jnp.where(cid == 0, KCF // GC, KCS // GC)

        def group(gi, _):
            base = row0 + gi * GC
            pltpu.sync_copy(src_ref.at[pl.ds(base, GC)], sidx)
            pltpu.sync_copy(dst_ref.at[pl.ds(base, GC)], didx)
            pltpu.sync_copy(g_ref.at[sidx.at[0]], buf0)

            def body(jj, _):
                j = 2 * jj
                d1 = pltpu.async_copy(g_ref.at[sidx.at[j + 1]], buf1, sem)
                scat(j, buf0)
                d1.wait()
                d2 = pltpu.async_copy(g_ref.at[sidx.at[j + 2]], buf0, sem)
                scat(j + 1, buf1)
                d2.wait()
                return 0

            lax.fori_loop(0, (GC - 2) // 2, body, 0)
            d = pltpu.async_copy(g_ref.at[sidx.at[GC - 1]], buf1, sem)
            scat(GC - 2, buf0)
            d.wait()
            scat(GC - 1, buf1)
            return 0

        lax.fori_loop(0, ngrp, group, 0)
        plsc.subcore_barrier()
        pltpu.sync_copy(acc.at[pl.ds(sid * RPS, RPS)],
                        out_ref.at[cid, pl.ds(sid * RPS, RPS)])

    return k(g_hbm, src_hbm, dst_hbm, zeros_hbm)


RB = 1024
GRID = NACC // RB


def _dinv_block(d0, d1):
    return lax.rsqrt(d0 + d1 + 1.0)


def _g1_body(x_ref, d0_ref, d1_ref, w_ref, o_ref):
    dinv = _dinv_block(d0_ref[:], d1_ref[:])
    o_ref[:] = dinv * jnp.dot(x_ref[:], w_ref[:],
                              preferred_element_type=jnp.float32)


def _mid_body(p0_ref, p1_ref, g_ref, d0_ref, d1_ref, b_ref, w_ref, o_ref):
    dinv = _dinv_block(d0_ref[:], d1_ref[:])
    h = jax.nn.relu(dinv * (p0_ref[:] + p1_ref[:] + g_ref[:]) + b_ref[:])
    o_ref[:] = dinv * jnp.dot(h, w_ref[:], preferred_element_type=jnp.float32)


def _head_body(p0_ref, p1_ref, g_ref, d0_ref, d1_ref, b_ref,
               wf1_ref, bf1_ref, wf2_ref, bf2_ref, o_ref):
    dinv = _dinv_block(d0_ref[:], d1_ref[:])
    h = jax.nn.relu(dinv * (p0_ref[:] + p1_ref[:] + g_ref[:]) + b_ref[:])
    f = jax.nn.relu(jnp.dot(h, wf1_ref[:], preferred_element_type=jnp.float32)
                    + bf1_ref[:])
    o_ref[:] = (jnp.dot(f, wf2_ref[:], preferred_element_type=jnp.float32)
                + bf2_ref[:])


def _row_spec(cols):
    return pl.BlockSpec((RB, cols), lambda i: (i, 0))


def _col_spec():
    return pl.BlockSpec((RB, 1), lambda i: (i, 0))


def _full_spec(r, c):
    return pl.BlockSpec((r, c), lambda i: (0, 0))


def _tc_g1(x_pad, d0, d1, W1):
    return pl.pallas_call(
        _g1_body,
        grid=(GRID,),
        in_specs=[_row_spec(D), _col_spec(), _col_spec(), _full_spec(D, D)],
        out_specs=_row_spec(D),
        out_shape=jax.ShapeDtypeStruct((NACC, D), jnp.float32),
    )(x_pad, d0, d1, W1)


def _tc_mid(p0, p1, g, d0, d1, b, W):
    return pl.pallas_call(
        _mid_body,
        grid=(GRID,),
        in_specs=[_row_spec(D), _row_spec(D), _row_spec(D),
                  _col_spec(), _col_spec(),
                  _full_spec(1, D), _full_spec(D, D)],
        out_specs=_row_spec(D),
        out_shape=jax.ShapeDtypeStruct((NACC, D), jnp.float32),
    )(p0, p1, g, d0, d1, b, W)


def _tc_head(p0, p1, g, d0, d1, b, Wf1, bf1, Wf2, bf2):
    return pl.pallas_call(
        _head_body,
        grid=(GRID,),
        in_specs=[_row_spec(D), _row_spec(D), _row_spec(D),
                  _col_spec(), _col_spec(),
                  _full_spec(1, D), _full_spec(D, FFN), _full_spec(1, FFN),
                  _full_spec(FFN, BOT), _full_spec(1, BOT)],
        out_specs=_row_spec(BOT),
        out_shape=jax.ShapeDtypeStruct((NACC, BOT), jnp.float32),
    )(p0, p1, g, d0, d1, b, Wf1, bf1, Wf2, bf2)


def kernel(x, edge_index, W1, b1, W2, b2, Wf1, bf1, Wf2, bf2):
    src = edge_index[0]
    dst = edge_index[1]
    npad = EPAD - E
    src_pad = jnp.concatenate(
        [src, jnp.zeros((npad,), jnp.int32)]).reshape(NW * KC, CH)
    dst_pad = jnp.concatenate(
        [dst, jnp.full((npad,), N, jnp.int32)]).reshape(NW * KC, CH)
    x_pad = jnp.concatenate(
        [x, jnp.zeros((NACC - N, D), jnp.float32)], axis=0)

    ones_ch = jnp.ones((CH,), jnp.float32)
    zeros_deg = jnp.zeros((RPS,), jnp.float32)
    zeros_conv = jnp.zeros((RPS, D), jnp.float32)

    deg_parts = _deg_kernel(dst_pad, ones_ch, zeros_deg)
    d0 = deg_parts[0].reshape(NACC, 1)
    d1 = deg_parts[1].reshape(NACC, 1)

    g1 = _tc_g1(x_pad, d0, d1, W1)
    acc1 = _conv_kernel(g1, src_pad, dst_pad, zeros_conv)
    g2 = _tc_mid(acc1[0], acc1[1], g1, d0, d1, b1.reshape(1, D), W2)
    acc2 = _conv_kernel(g2, src_pad, dst_pad, zeros_conv)
    out = _tc_head(acc2[0], acc2[1], g2, d0, d1, b2.reshape(1, D),
                   Wf1, bf1.reshape(1, FFN), Wf2, bf2.reshape(1, BOT))
    return out[:N]


# speedup vs baseline: 10.3001x; 1.0352x over previous
import functools

import jax
import jax.numpy as jnp
from jax import lax
from jax.experimental import pallas as pl
from jax.experimental.pallas import tpu as pltpu
from jax.experimental.pallas import tpu_sc as plsc

N = 10000
E = 320000
D = 128
FFN = 256
BOT = 64

NC = 2
NS = 16
NW = NC * NS
CH = 128
KC = 80
GC = 16
KCF = 128
KCS = 32
EPW = CH * KC
EPAD = NW * EPW
NACC = 10240
RPS = NACC // NS


def _sc_mesh():
    return plsc.VectorSubcoreMesh(core_axis_name="c", subcore_axis_name="s")


def _deg_kernel(dst_hbm, ones_hbm, zeros_hbm):
    @functools.partial(
        pl.kernel,
        out_type=jax.ShapeDtypeStruct((NC, NACC), jnp.float32),
        mesh=_sc_mesh(),
        scratch_types=[
            pltpu.VMEM_SHARED((NACC,), jnp.float32),
            pltpu.VMEM((KC, CH), jnp.int32),
            pltpu.VMEM((CH,), jnp.float32),
        ],
    )
    def k(dst_ref, ones_ref, zeros_ref, out_ref, acc, idx, ones_v):
        cid = lax.axis_index("c")
        sid = lax.axis_index("s")
        wid = cid * NS + sid
        pltpu.sync_copy(zeros_ref, acc.at[pl.ds(sid * RPS, RPS)])
        pltpu.sync_copy(ones_ref, ones_v)
        pltpu.sync_copy(dst_ref.at[pl.ds(wid * KC, KC)], idx)
        plsc.subcore_barrier()

        def body(j, _):
            pltpu.sync_copy(ones_v, acc.at[idx.at[j]], add=True)
            return 0

        lax.fori_loop(0, KC, body, 0)
        plsc.subcore_barrier()
        pltpu.sync_copy(acc.at[pl.ds(sid * RPS, RPS)],
                        out_ref.at[cid, pl.ds(sid * RPS, RPS)])

    return k(dst_hbm, ones_hbm, zeros_hbm)


def _conv_kernel(g_hbm, src_hbm, dst_hbm, zeros_hbm):
    @functools.partial(
        pl.kernel,
        out_type=jax.ShapeDtypeStruct((NC, NACC, D), jnp.float32),
        mesh=_sc_mesh(),
        scratch_types=[
            pltpu.VMEM_SHARED((NACC, D), jnp.float32),
            pltpu.VMEM((GC, CH), jnp.int32),
            pltpu.VMEM((GC, CH), jnp.int32),
            pltpu.VMEM((CH, D), jnp.float32),
            pltpu.VMEM((CH, D), jnp.float32),
            pltpu.SemaphoreType.DMA,
        ],
    )
    def k(g_ref, src_ref, dst_ref, zeros_ref, out_ref,
          acc, sidx, didx, buf0, buf1, sem):
        cid = lax.axis_index("c")
        sid = lax.axis_index("s")
        wid = cid * NS + sid
        for t in range(10):
            pltpu.sync_copy(zeros_ref.at[pl.ds(t * 64, 64)],
                            acc.at[pl.ds(sid * RPS + t * 64, 64)])
        plsc.subcore_barrier()

        def scat(j, buf):
            pltpu.sync_copy(buf, acc.at[didx.at[j]], add=True)

        row0 = jnp.where(cid == 0, sid * KCF, NS * KCF + sid * KCS)
        ngrp = jnp.where(cid == 0, KCF // GC, KCS // GC)

        def group(gi, _):
            base = row0 + gi * GC
            pltpu.sync_copy(src_ref.at[pl.ds(base, GC)], sidx)
            pltpu.sync_copy(dst_ref.at[pl.ds(base, GC)], didx)
            pltpu.sync_copy(g_ref.at[sidx.at[0]], buf0)

            def body(jj, _):
                j = 2 * jj
                d1 = pltpu.async_copy(g_ref.at[sidx.at[j + 1]], buf1, sem)
                scat(j, buf0)
                d1.wait()
                d2 = pltpu.async_copy(g_ref.at[sidx.at[j + 2]], buf0, sem)
                scat(j + 1, buf1)
                d2.wait()
                return 0

            lax.fori_loop(0, (GC - 2) // 2, body, 0)
            d = pltpu.async_copy(g_ref.at[sidx.at[GC - 1]], buf1, sem)
            scat(GC - 2, buf0)
            d.wait()
            scat(GC - 1, buf1)
            return 0

        lax.fori_loop(0, ngrp, group, 0)
        plsc.subcore_barrier()
        pltpu.sync_copy(acc.at[pl.ds(sid * RPS, RPS)],
                        out_ref.at[cid, pl.ds(sid * RPS, RPS)])

    return k(g_hbm, src_hbm, dst_hbm, zeros_hbm)


RB = 1024
GRID = NACC // RB


def _dinv_block(d0, d1):
    return lax.rsqrt(d0 + d1 + 1.0)


def _g1_body(x_ref, d0_ref, d1_ref, w_ref, o_ref):
    dinv = _dinv_block(d0_ref[:], d1_ref[:])
    o_ref[:] = dinv * jnp.dot(x_ref[:], w_ref[:],
                              preferred_element_type=jnp.float32)


def _mid_body(p0_ref, p1_ref, g_ref, d0_ref, d1_ref, b_ref, w_ref, o_ref):
    dinv = _dinv_block(d0_ref[:], d1_ref[:])
    h = jax.nn.relu(dinv * (p0_ref[:] + p1_ref[:] + g_ref[:]) + b_ref[:])
    o_ref[:] = dinv * jnp.dot(h, w_ref[:], preferred_element_type=jnp.float32)


def _head_body(p0_ref, p1_ref, g_ref, d0_ref, d1_ref, b_ref,
               wf1_ref, bf1_ref, wf2_ref, bf2_ref, o_ref):
    dinv = _dinv_block(d0_ref[:], d1_ref[:])
    h = jax.nn.relu(dinv * (p0_ref[:] + p1_ref[:] + g_ref[:]) + b_ref[:])
    f = jax.nn.relu(jnp.dot(h, wf1_ref[:], preferred_element_type=jnp.float32)
                    + bf1_ref[:])
    o_ref[:] = (jnp.dot(f, wf2_ref[:], preferred_element_type=jnp.float32)
                + bf2_ref[:])


def _row_spec(cols):
    return pl.BlockSpec((RB, cols), lambda i: (i, 0))


def _col_spec():
    return pl.BlockSpec((RB, 1), lambda i: (i, 0))


def _full_spec(r, c):
    return pl.BlockSpec((r, c), lambda i: (0, 0))


def _tc_g1(x_pad, d0, d1, W1):
    return pl.pallas_call(
        _g1_body,
        grid=(GRID,),
        in_specs=[_row_spec(D), _col_spec(), _col_spec(), _full_spec(D, D)],
        out_specs=_row_spec(D),
        out_shape=jax.ShapeDtypeStruct((NACC, D), jnp.float32),
    )(x_pad, d0, d1, W1)


def _tc_mid(p0, p1, g, d0, d1, b, W):
    return pl.pallas_call(
        _mid_body,
        grid=(GRID,),
        in_specs=[_row_spec(D), _row_spec(D), _row_spec(D),
                  _col_spec(), _col_spec(),
                  _full_spec(1, D), _full_spec(D, D)],
        out_specs=_row_spec(D),
        out_shape=jax.ShapeDtypeStruct((NACC, D), jnp.float32),
    )(p0, p1, g, d0, d1, b, W)


def _tc_head(p0, p1, g, d0, d1, b, Wf1, bf1, Wf2, bf2):
    return pl.pallas_call(
        _head_body,
        grid=(GRID,),
        in_specs=[_row_spec(D), _row_spec(D), _row_spec(D),
                  _col_spec(), _col_spec(),
                  _full_spec(1, D), _full_spec(D, FFN), _full_spec(1, FFN),
                  _full_spec(FFN, BOT), _full_spec(1, BOT)],
        out_specs=_row_spec(BOT),
        out_shape=jax.ShapeDtypeStruct((NACC, BOT), jnp.float32),
    )(p0, p1, g, d0, d1, b, Wf1, bf1, Wf2, bf2)


def kernel(x, edge_index, W1, b1, W2, b2, Wf1, bf1, Wf2, bf2):
    src = edge_index[0]
    dst = edge_index[1]
    npad = EPAD - E
    src_pad = jnp.concatenate(
        [src, jnp.zeros((npad,), jnp.int32)]).reshape(NW * KC, CH)
    dst_pad = jnp.concatenate(
        [dst, jnp.full((npad,), N, jnp.int32)]).reshape(NW * KC, CH)
    x_pad = jnp.concatenate(
        [x, jnp.zeros((NACC - N, D), jnp.float32)], axis=0)

    ones_ch = jnp.ones((CH,), jnp.float32)
    zeros_deg = jnp.zeros((RPS,), jnp.float32)
    zeros_conv = jnp.zeros((RPS, D), jnp.float32)

    deg_parts = _deg_kernel(dst_pad, ones_ch, zeros_deg)
    d0 = deg_parts[0].reshape(NACC, 1)
    d1 = deg_parts[1].reshape(NACC, 1)

    g1 = _tc_g1(x_pad, d0, d1, W1)
    acc1 = _conv_kernel(g1, src_pad, dst_pad, zeros_conv)
    g2 = _tc_mid(acc1[0], acc1[1], g1, d0, d1, b1.reshape(1, D), W2)
    acc2 = _conv_kernel(g2, src_pad, dst_pad, zeros_conv)
    out = _tc_head(acc2[0], acc2[1], g2, d0, d1, b2.reshape(1, D),
                   Wf1, bf1.reshape(1, FFN), Wf2, bf2.reshape(1, BOT))
    return out[:N]
